# baseline (speedup 1.0000x reference)
import jax
import jax.numpy as jnp
from jax.experimental import pallas as pl
from jax.experimental.pallas import tpu as pltpu

_BT = 512


def _fused_kernel(mat_o_ref, mat_r_ref, src_o_ref, src_r_ref,
                  tgt_o_ref, tgt_r_ref, w_o_ref, w_r_ref,
                  b_o_ref, b_r_ref, out_o_ref, out_r_ref):
    def one(mat_ref, src_ref, tgt_ref, w_ref, b_ref, out_ref):
        mat = mat_ref[...]
        m = mat > 0
        cnt = jnp.sum(m.astype(jnp.float32), axis=1, keepdims=True)
        acc = jnp.dot(m.astype(jnp.bfloat16), src_ref[...],
                      preferred_element_type=jnp.float32)
        msg = jnp.where(cnt > 0, acc / jnp.maximum(cnt, 1.0), 0.0)
        h = jnp.maximum(msg, 0.0)
        upd = jnp.dot(h, w_ref[...], preferred_element_type=jnp.float32)
        out_ref[...] = tgt_ref[...] + upd + b_ref[...]

    one(mat_o_ref, src_r_ref, tgt_o_ref, w_o_ref, b_o_ref, out_o_ref)
    one(mat_r_ref, src_o_ref, tgt_r_ref, w_r_ref, b_r_ref, out_r_ref)


def kernel(feature_obj, feature_region, mat_object, mat_region,
           W_r2o, b_r2o, W_o2r, b_o2r):
    T, S = mat_object.shape
    D = feature_obj.shape[1]
    big = pl.BlockSpec((_BT, S), lambda i: (i, 0))
    src = pl.BlockSpec((S, D), lambda i: (0, 0))
    row = pl.BlockSpec((_BT, D), lambda i: (i, 0))
    wsp = pl.BlockSpec((D, D), lambda i: (0, 0))
    bsp = pl.BlockSpec((1, D), lambda i: (0, 0))
    out_o, out_r = pl.pallas_call(
        _fused_kernel,
        grid=(T // _BT,),
        in_specs=[big, big, src, src, row, row, wsp, wsp, bsp, bsp],
        out_specs=[row, row],
        out_shape=[jax.ShapeDtypeStruct((T, D), jnp.float32),
                   jax.ShapeDtypeStruct((T, D), jnp.float32)],
        compiler_params=pltpu.CompilerParams(
            dimension_semantics=("parallel",)),
    )(mat_object, mat_region,
      feature_obj.astype(jnp.bfloat16), feature_region.astype(jnp.bfloat16),
      feature_obj, feature_region, W_r2o.T, W_o2r.T,
      b_r2o.reshape(1, -1), b_o2r.reshape(1, -1))
    return (out_o, out_r)
